# Initial kernel scaffold; baseline (speedup 1.0000x reference)
import jax
import jax.numpy as jnp
from jax.experimental import pallas as pl


def kernel(x, edge_index, batch, interface_pos, graph_num, W1, b1, W2, b2, W3, b3, Wn, bn, K1, bk1, K2, bk2, L0w, L0b, L1w, L1b, L2w, L2b):
    raise NotImplementedError("write your pallas kernel here")


# speedup vs baseline: 8.0579x; 8.0579x over previous
import functools

import jax
import jax.numpy as jnp
from jax import lax
from jax.experimental import pallas as pl
from jax.experimental.pallas import tpu as pltpu
from jax.experimental.pallas import tpu_sc as plsc

N = 10000
NP = 10240
DF = 128
NG = 32
KTOP = 100
M = 4000
MP = 4096
SLOTSP = NG * 104
EB = 128
NSUB = 16
NCORE = 2
NTILE = NSUB * NCORE
PSUB = NP // NSUB
SENT = N
BM = 1024

_mesh = functools.partial(
    plsc.VectorSubcoreMesh, core_axis_name="c", subcore_axis_name="s",
    num_cores=NCORE, num_subcores=NSUB)


def _sc_scatter_accum(nchunks, et, gather, interpret=False):
  @functools.partial(
      pl.kernel,
      out_type=jax.ShapeDtypeStruct((NCORE, NP, DF), jnp.float32),
      mesh=_mesh(),
      interpret=interpret,
      compiler_params=pltpu.CompilerParams(needs_layout_passes=False),
      scratch_types=[
          pltpu.VMEM_SHARED((NP, DF), jnp.float32),
          pltpu.VMEM((EB,), jnp.int32),
          pltpu.VMEM((EB,), jnp.int32),
          pltpu.VMEM((EB, DF), jnp.float32),
          pltpu.VMEM((16, DF), jnp.float32),
          pltpu.SemaphoreType.DMA,
      ],
  )
  def body(tbl_hbm, src_hbm, dst_hbm, out_hbm, acc, sidx, didx, rows, zb, sem):
    c = lax.axis_index("c")
    s = lax.axis_index("s")
    wid = c * NSUB + s

    def zrow(k, _):
      def zcol(j, _):
        zb[k, pl.ds(j * 16, 16)] = jnp.zeros((16,), jnp.float32)
        return 0
      return lax.fori_loop(0, DF // 16, zcol, 0)
    lax.fori_loop(0, 16, zrow, 0)

    if not gather:
      def orow(k, _):
        def ocol(j, _):
          rows[k, pl.ds(j * 16, 16)] = jnp.ones((16,), jnp.float32)
          return 0
        return lax.fori_loop(0, DF // 16, ocol, 0)
      lax.fori_loop(0, EB, orow, 0)

    def zslab(k, _):
      pltpu.sync_copy(zb, acc.at[pl.ds(s * PSUB + k * 16, 16)])
      return 0
    lax.fori_loop(0, PSUB // 16, zslab, 0)
    plsc.subcore_barrier()

    def chunk(i, _):
      base = wid * et + i * EB
      pltpu.sync_copy(dst_hbm.at[pl.ds(base, EB)], didx)
      if gather:
        pltpu.sync_copy(src_hbm.at[pl.ds(base, EB)], sidx)
        pltpu.async_copy(tbl_hbm.at[sidx], rows, sem).wait()
      pltpu.sync_copy(rows, acc.at[didx], add=True)
      return 0
    lax.fori_loop(0, nchunks, chunk, 0)
    plsc.subcore_barrier()

    pltpu.sync_copy(acc.at[pl.ds(s * PSUB, PSUB)],
                    out_hbm.at[c, pl.ds(s * PSUB, PSUB)])

  return body


def _sc_gather_rows(n_idx, interpret=False):
  per = n_idx // NTILE

  @functools.partial(
      pl.kernel,
      out_type=jax.ShapeDtypeStruct((n_idx, DF), jnp.float32),
      mesh=_mesh(),
      interpret=interpret,
      compiler_params=pltpu.CompilerParams(needs_layout_passes=False),
      scratch_types=[
          pltpu.VMEM((per,), jnp.int32),
          pltpu.VMEM((per, DF), jnp.float32),
          pltpu.SemaphoreType.DMA,
      ],
  )
  def body(tbl_hbm, idx_hbm, out_hbm, idxv, rows, sem):
    c = lax.axis_index("c")
    s = lax.axis_index("s")
    base = (c * NSUB + s) * per
    pltpu.sync_copy(idx_hbm.at[pl.ds(base, per)], idxv)
    pltpu.async_copy(tbl_hbm.at[idxv], rows, sem).wait()
    pltpu.sync_copy(rows, out_hbm.at[pl.ds(base, per)])

  return body


def _sc_slots(interpret=False):
  @functools.partial(
      pl.kernel,
      out_type=jax.ShapeDtypeStruct((SLOTSP,), jnp.int32),
      mesh=_mesh(),
      interpret=interpret,
      compiler_params=pltpu.CompilerParams(needs_layout_passes=False),
      scratch_types=[
          pltpu.VMEM((MP,), jnp.int32),
          pltpu.VMEM((MP,), jnp.int32),
          pltpu.VMEM((MP,), jnp.int32),
          pltpu.VMEM((SLOTSP,), jnp.int32),
      ],
  )
  def body(rank_hbm, sb_hbm, ip_hbm, out_hbm, rnk, sbv, ipv, slots):
    c = lax.axis_index("c")
    s = lax.axis_index("s")

    @pl.when(jnp.logical_and(c == 0, s == 0))
    def _():
      pltpu.sync_copy(rank_hbm, rnk)
      pltpu.sync_copy(sb_hbm, sbv)
      pltpu.sync_copy(ip_hbm, ipv)

      def init(k, _):
        slots[pl.ds(k * 16, 16)] = jnp.full((16,), SENT, jnp.int32)
        return 0
      lax.fori_loop(0, SLOTSP // 16, init, 0)

      def step(i, _):
        r = rnk[pl.ds(i * 16, 16)]
        g = sbv[pl.ds(i * 16, 16)]
        node = ipv[pl.ds(i * 16, 16)]
        ok = jnp.logical_and(g < NG, r < KTOP)
        slot = jnp.where(ok, g * KTOP + r, SLOTSP - 1)
        val = jnp.where(ok, node, SENT)
        plsc.store_scatter(slots, [slot], val)
        return 0
      lax.fori_loop(0, MP // 16, step, 0)
      pltpu.sync_copy(slots, out_hbm)

  return body


def _tc_g1(interpret=False):
  def body(x_ref, w_ref, degp_ref, g_ref):
    deg = degp_ref[0] + degp_ref[1] + 1.0
    dinv = lax.rsqrt(deg)
    g_ref[...] = dinv * jnp.dot(x_ref[...], w_ref[...],
                                preferred_element_type=jnp.float32,
                 precision=lax.Precision.HIGHEST)

  return pl.pallas_call(
      body,
      grid=(NP // BM,),
      in_specs=[
          pl.BlockSpec((BM, DF), lambda i: (i, 0)),
          pl.BlockSpec((DF, DF), lambda i: (0, 0)),
          pl.BlockSpec((2, BM, 1), lambda i: (0, i, 0)),
      ],
      out_specs=pl.BlockSpec((BM, DF), lambda i: (i, 0)),
      out_shape=jax.ShapeDtypeStruct((NP, DF), jnp.float32),
      interpret=interpret,
  )


def _tc_comb(bcast, interpret=False):
  def body(sp_ref, g_ref, degp_ref, b_ref, w_ref, h_ref, gn_ref):
    i = pl.program_id(0)
    deg = degp_ref[0] + degp_ref[1] + 1.0
    dinv = lax.rsqrt(deg)
    t = jnp.tanh(dinv * (sp_ref[0] + sp_ref[1] + g_ref[...]) + b_ref[...])
    rows = i * BM + lax.broadcasted_iota(jnp.int32, (BM, 1), 0)
    h = jnp.where(rows < N, t, 0.0)
    h_ref[...] = h
    p = jnp.dot(h, w_ref[...], preferred_element_type=jnp.float32,
                 precision=lax.Precision.HIGHEST)
    if bcast:
      p = jnp.broadcast_to(p, (BM, DF))
    gn_ref[...] = dinv * p

  wcols = 1 if bcast else DF
  return pl.pallas_call(
      body,
      grid=(NP // BM,),
      in_specs=[
          pl.BlockSpec((2, BM, DF), lambda i: (0, i, 0)),
          pl.BlockSpec((BM, DF), lambda i: (i, 0)),
          pl.BlockSpec((2, BM, 1), lambda i: (0, i, 0)),
          pl.BlockSpec((1, DF), lambda i: (0, 0)),
          pl.BlockSpec((DF, wcols), lambda i: (0, 0)),
      ],
      out_specs=[
          pl.BlockSpec((BM, DF), lambda i: (i, 0)),
          pl.BlockSpec((BM, DF), lambda i: (i, 0)),
      ],
      out_shape=[
          jax.ShapeDtypeStruct((NP, DF), jnp.float32),
          jax.ShapeDtypeStruct((NP, DF), jnp.float32),
      ],
      interpret=interpret,
  )


def _tc_final(interpret=False):
  def body(sp_ref, g4_ref, degp_ref, bn_ref, batch_ref, h1_ref, h2_ref,
           h3_ref, ka_ref, kb_ref, kc_ref, kd_ref, t_ref):
    i = pl.program_id(0)
    deg = degp_ref[0] + degp_ref[1] + 1.0
    dinv = lax.rsqrt(deg)
    t = jnp.tanh(dinv * (sp_ref[0] + sp_ref[1] + g4_ref[...]) + bn_ref[...])
    rows = i * BM + lax.broadcasted_iota(jnp.int32, (BM, 1), 0)
    h4 = jnp.where(rows < N, t, 0.0)
    q = (jnp.dot(h1_ref[...], ka_ref[...], preferred_element_type=jnp.float32,
                 precision=lax.Precision.HIGHEST)
         + jnp.dot(h2_ref[...], kb_ref[...], preferred_element_type=jnp.float32,
                 precision=lax.Precision.HIGHEST)
         + jnp.dot(h3_ref[...], kc_ref[...], preferred_element_type=jnp.float32,
                 precision=lax.Precision.HIGHEST)
         + h4 * kd_ref[...])
    bb = lax.bitcast_convert_type(batch_ref[...], jnp.float32)
    t_ref[...] = jnp.concatenate(
        [q, h4, bb, jnp.zeros((BM, DF - 20), jnp.float32)], axis=1)

  return pl.pallas_call(
      body,
      grid=(NP // BM,),
      in_specs=[
          pl.BlockSpec((2, BM, 1), lambda i: (0, i, 0)),
          pl.BlockSpec((BM, 1), lambda i: (i, 0)),
          pl.BlockSpec((2, BM, 1), lambda i: (0, i, 0)),
          pl.BlockSpec((1, 1), lambda i: (0, 0)),
          pl.BlockSpec((BM, 1), lambda i: (i, 0)),
          pl.BlockSpec((BM, DF), lambda i: (i, 0)),
          pl.BlockSpec((BM, DF), lambda i: (i, 0)),
          pl.BlockSpec((BM, DF), lambda i: (i, 0)),
          pl.BlockSpec((DF, 18), lambda i: (0, 0)),
          pl.BlockSpec((DF, 18), lambda i: (0, 0)),
          pl.BlockSpec((DF, 18), lambda i: (0, 0)),
          pl.BlockSpec((1, 18), lambda i: (0, 0)),
      ],
      out_specs=pl.BlockSpec((BM, DF), lambda i: (i, 0)),
      out_shape=jax.ShapeDtypeStruct((NP, DF), jnp.float32),
      interpret=interpret,
  )


def _tc_rank(interpret=False):
  BI = 128

  def body(vt_ref, vc_ref, rank_ref, sb_ref):
    i0 = pl.program_id(0) * BI
    vi = vc_ref[:, 0:1]
    sbi = lax.bitcast_convert_type(vc_ref[:, 1:2], jnp.int32)
    vj = vt_ref[0:1, :]
    sbj = lax.bitcast_convert_type(vt_ref[1:2, :], jnp.int32)
    iidx = i0 + lax.broadcasted_iota(jnp.int32, (BI, 1), 0)
    jidx = lax.broadcasted_iota(jnp.int32, (1, MP), 1)
    same = sbj == sbi
    gt = vj > vi
    tie = jnp.logical_and(vj == vi, jidx < iidx)
    cnt = jnp.sum(
        jnp.where(jnp.logical_and(same, jnp.logical_or(gt, tie)), 1, 0),
        axis=1)
    rank_ref[...] = cnt.reshape(BI, 1)
    sb_ref[...] = sbi

  return pl.pallas_call(
      body,
      grid=(MP // BI,),
      in_specs=[
          pl.BlockSpec((2, MP), lambda i: (0, 0)),
          pl.BlockSpec((BI, 2), lambda i: (i, 0)),
      ],
      out_specs=[
          pl.BlockSpec((BI, 1), lambda i: (i, 0)),
          pl.BlockSpec((BI, 1), lambda i: (i, 0)),
      ],
      out_shape=[
          jax.ShapeDtypeStruct((MP, 1), jnp.int32),
          jax.ShapeDtypeStruct((MP, 1), jnp.int32),
      ],
      interpret=interpret,
  )


def _tc_head(interpret=False):
  def body(rsel_ref, bk1_ref, k2m_ref, bk2_ref, l0w_ref, l0b_ref,
           l1w_ref, l1b_ref, l2w_ref, l2b_ref, out_ref):
    z1 = jax.nn.relu(rsel_ref[0:NG * KTOP, :] + bk1_ref[...])
    z1 = z1.reshape(NG, KTOP // 2, 2, DF)
    pool = jnp.max(z1, axis=2)
    u = jnp.concatenate([pool[:, t:t + 46, :] for t in range(5)], axis=2)
    u = u.reshape(NG * 46, 5 * DF)
    z2 = jax.nn.relu(jnp.dot(u, k2m_ref[...],
                             preferred_element_type=jnp.float32,
                 precision=lax.Precision.HIGHEST)
                     + bk2_ref[...])
    z2r = z2.reshape(NG, 46, 36)
    acc = jnp.zeros((NG, 512), jnp.float32)
    for p in range(46):
      acc = acc + jnp.dot(z2r[:, p, :], l0w_ref[p],
                          preferred_element_type=jnp.float32,
                 precision=lax.Precision.HIGHEST)
    z = jax.nn.relu(acc + l0b_ref[...])
    z = jax.nn.relu(jnp.dot(z, l1w_ref[...],
                            preferred_element_type=jnp.float32,
                 precision=lax.Precision.HIGHEST) + l1b_ref[...])
    out_ref[...] = jnp.dot(z, l2w_ref[...],
                           preferred_element_type=jnp.float32,
                 precision=lax.Precision.HIGHEST) + l2b_ref[...]

  return pl.pallas_call(
      body,
      out_shape=jax.ShapeDtypeStruct((NG, 1), jnp.float32),
      interpret=interpret,
  )


def kernel(x, edge_index, batch, interface_pos, graph_num, W1, b1, W2, b2,
           W3, b3, Wn, bn, K1, bk1, K2, bk2, L0w, L0b, L1w, L1b, L2w, L2b):
  E = edge_index.shape[1]
  et = -(-E // NTILE // EB) * EB
  EP = et * NTILE
  nchunks = et // EB

  src_p = jnp.concatenate(
      [edge_index[0], jnp.full((EP - E,), SENT, jnp.int32)])
  dst_p = jnp.concatenate(
      [edge_index[1], jnp.full((EP - E,), SENT, jnp.int32)])
  x_p = jnp.pad(x, ((0, NP - N), (0, 0)))
  batch_p = jnp.pad(batch, (0, NP - N), constant_values=127).reshape(NP, 1)
  ip_p = jnp.pad(interface_pos, (0, MP - M), constant_values=SENT)

  deg_k = _sc_scatter_accum(nchunks, et, gather=False)
  scat = _sc_scatter_accum(nchunks, et, gather=True)

  degp = deg_k(jnp.zeros((NP, DF), jnp.float32), src_p, dst_p)
  degc = degp[:, :, 0:1]

  g1 = _tc_g1()(x_p, W1, degc)
  s1 = scat(g1, src_p, dst_p)
  h1, g2 = _tc_comb(False)(s1, g1, degc, b1.reshape(1, DF), W2)
  s2 = scat(g2, src_p, dst_p)
  h2, g3 = _tc_comb(False)(s2, g2, degc, b2.reshape(1, DF), W3)
  s3 = scat(g3, src_p, dst_p)
  h3, g4 = _tc_comb(True)(s3, g3, degc, b3.reshape(1, DF), Wn)
  s4 = scat(g4, src_p, dst_p)

  K1m = K1[:, 0, :]
  ka = K1m[:, 0:128].T
  kb = K1m[:, 128:256].T
  kc = K1m[:, 256:384].T
  kd = K1m[:, 384].reshape(1, 18)

  tbl = _tc_final()(s4[:, :, 0:1], g4[:, 0:1], degc, bn.reshape(1, 1),
                    batch_p, h1, h2, h3, ka, kb, kc, kd)

  vbsel = _sc_gather_rows(MP)(tbl, ip_p)
  vbc = vbsel[:, 18:20]
  rank, sbsel = _tc_rank()(vbc.T, vbc)
  slots = _sc_slots()(rank.reshape(MP), sbsel.reshape(MP), ip_p)
  rsel = _sc_gather_rows(SLOTSP)(tbl, slots)

  k2m = jnp.pad(jnp.transpose(K2, (2, 1, 0)),
                ((0, 0), (0, DF - 18), (0, 0))).reshape(5 * DF, 36)
  l0wp = L0w.reshape(36, 46, -1).transpose(1, 0, 2)
  bk1p = jnp.pad(bk1.reshape(1, 18), ((0, 0), (0, DF - 18)),
                 constant_values=-1e30)

  return _tc_head()(rsel, bk1p, k2m, bk2.reshape(1, 36), l0wp,
                    L0b.reshape(1, -1), L1w, L1b.reshape(1, -1),
                    L2w, L2b.reshape(1, -1))
